# baseline (speedup 1.0000x reference)
import functools

import jax
import jax.numpy as jnp
from jax.experimental import pallas as pl
from jax.experimental.pallas import tpu as pltpu

S, D, H, E, TOPK, F = 2048, 1024, 64, 16, 2, 64

BS_GATE = 256
BS_EXP = 256


def _gate_kernel(x_ref, wout_ref, wgin_ref, wglin_ref, topi_ref, w_ref, v_ref):
    i = pl.program_id(0)
    nblk = pl.num_programs(0)

    @pl.when(i == 0)
    def _():
        v_ref[...] = jnp.zeros_like(v_ref)

    v_ref[...] += jnp.sum(x_ref[...] * wout_ref[...], axis=0, keepdims=True)

    @pl.when(i == nblk - 1)
    def _():
        u = jnp.dot(v_ref[...], wgin_ref[...],
                    preferred_element_type=jnp.float32)
        g = jnp.dot(u, wglin_ref[...],
                    preferred_element_type=jnp.float32)
        iota = jax.lax.broadcasted_iota(jnp.int32, (1, E), 1)
        m0 = jnp.max(g)
        i0 = jnp.min(jnp.where(g == m0, iota, E))
        g2 = jnp.where(iota == i0, -jnp.inf, g)
        m1 = jnp.max(g2)
        i1 = jnp.min(jnp.where(g2 == m1, iota, E))
        e1 = jnp.exp(m1 - m0)
        denom = 1.0 + e1
        topi_ref[0] = i0
        topi_ref[1] = i1
        w_ref[0] = 1.0 / denom
        w_ref[1] = e1 / denom


def _expert_kernel(topi_ref, x_ref, w0_ref, w1_ref, gw_ref, out_ref):
    xb = x_ref[...]
    z0 = jnp.dot(xb, w0_ref[0], preferred_element_type=jnp.float32)
    z1 = jnp.dot(xb, w1_ref[0], preferred_element_type=jnp.float32)

    def norm_gelu(z):
        nrm = jnp.sqrt(jnp.sum(z * z, axis=-1, keepdims=True))
        zn = z / jnp.maximum(nrm, 1e-12)
        return 0.5 * zn * (1.0 + jax.lax.erf(zn * 0.7071067811865476))

    out_ref[...] = gw_ref[0] * norm_gelu(z0) + gw_ref[1] * norm_gelu(z1)


@jax.jit
def kernel(x, W_gate_in, W_gate_lin, W_gate_out, W_experts):
    topi, gw = pl.pallas_call(
        _gate_kernel,
        grid=(S // BS_GATE,),
        in_specs=[
            pl.BlockSpec((BS_GATE, D), lambda i: (i, 0)),
            pl.BlockSpec((BS_GATE, 1), lambda i: (i, 0)),
            pl.BlockSpec((D, H), lambda i: (0, 0)),
            pl.BlockSpec((H, E), lambda i: (0, 0)),
        ],
        out_specs=[
            pl.BlockSpec(memory_space=pltpu.SMEM),
            pl.BlockSpec(memory_space=pltpu.SMEM),
        ],
        out_shape=[
            jax.ShapeDtypeStruct((TOPK,), jnp.int32),
            jax.ShapeDtypeStruct((TOPK,), jnp.float32),
        ],
        scratch_shapes=[pltpu.VMEM((1, D), jnp.float32)],
    )(x, W_gate_out, W_gate_in, W_gate_lin)

    y = pl.pallas_call(
        _expert_kernel,
        grid_spec=pltpu.PrefetchScalarGridSpec(
            num_scalar_prefetch=1,
            grid=(S // BS_EXP,),
            in_specs=[
                pl.BlockSpec((BS_EXP, D), lambda i, topi: (i, 0)),
                pl.BlockSpec((1, D, F), lambda i, topi: (topi[0], 0, 0)),
                pl.BlockSpec((1, D, F), lambda i, topi: (topi[1], 0, 0)),
                pl.BlockSpec(memory_space=pltpu.SMEM),
            ],
            out_specs=pl.BlockSpec((BS_EXP, F), lambda i, topi: (i, 0)),
        ),
        out_shape=jax.ShapeDtypeStruct((S, F), jnp.float32),
    )(topi, x, W_experts, W_experts, gw)
    return y


# speedup vs baseline: 1.1218x; 1.1218x over previous
import jax
import jax.numpy as jnp
from jax.experimental import pallas as pl
from jax.experimental.pallas import tpu as pltpu

S, D, H, E, TOPK, F = 2048, 1024, 64, 16, 2, 64

BS = 512
NBLK = S // BS


def _moe_kernel(x_ref, wout_ref, wgin_ref, wglin_ref, wexp_hbm, out_ref,
                v_ref, w0_ref, w1_ref, gw_ref, sems):
    p = pl.program_id(0)
    i = pl.program_id(1)

    @pl.when(p == 0)
    def _gate_phase():
        @pl.when(i == 0)
        def _():
            v_ref[...] = jnp.zeros_like(v_ref)

        v_ref[...] += jax.lax.dot_general(
            wout_ref[...], x_ref[...],
            dimension_numbers=(((0,), (0,)), ((), ())),
            preferred_element_type=jnp.float32)

        @pl.when(i == NBLK - 1)
        def _finish_gate():
            u = jnp.dot(v_ref[...], wgin_ref[...],
                        preferred_element_type=jnp.float32)
            g = jnp.dot(u, wglin_ref[...],
                        preferred_element_type=jnp.float32)
            iota = jax.lax.broadcasted_iota(jnp.int32, (1, E), 1)
            m0 = jnp.max(g)
            i0 = jnp.min(jnp.where(g == m0, iota, E))
            g2 = jnp.where(iota == i0, -jnp.inf, g)
            m1 = jnp.max(g2)
            i1 = jnp.min(jnp.where(g2 == m1, iota, E))
            e1 = jnp.exp(m1 - m0)
            denom = 1.0 + e1
            gw_ref[0] = 1.0 / denom
            gw_ref[1] = e1 / denom
            pltpu.make_async_copy(wexp_hbm.at[i0], w0_ref, sems.at[0]).start()
            pltpu.make_async_copy(wexp_hbm.at[i1], w1_ref, sems.at[1]).start()

    @pl.when(p == 1)
    def _expert_phase():
        @pl.when(i == 0)
        def _():
            pltpu.make_async_copy(wexp_hbm.at[0], w0_ref, sems.at[0]).wait()
            pltpu.make_async_copy(wexp_hbm.at[0], w1_ref, sems.at[1]).wait()

        xb = x_ref[...]
        z0 = jnp.dot(xb, w0_ref[...], preferred_element_type=jnp.float32)
        z1 = jnp.dot(xb, w1_ref[...], preferred_element_type=jnp.float32)

        def norm_gelu(z):
            nrm = jnp.sqrt(jnp.sum(z * z, axis=-1, keepdims=True))
            zn = z / jnp.maximum(nrm, 1e-12)
            return 0.5 * zn * (1.0 + jax.lax.erf(zn * 0.7071067811865476))

        out_ref[...] = gw_ref[0] * norm_gelu(z0) + gw_ref[1] * norm_gelu(z1)


@jax.jit
def kernel(x, W_gate_in, W_gate_lin, W_gate_out, W_experts):
    return pl.pallas_call(
        _moe_kernel,
        grid=(2, NBLK),
        in_specs=[
            pl.BlockSpec((BS, D), lambda p, i: (i, 0)),
            pl.BlockSpec((BS, 1), lambda p, i: (i, 0)),
            pl.BlockSpec((D, H), lambda p, i: (0, 0)),
            pl.BlockSpec((H, E), lambda p, i: (0, 0)),
            pl.BlockSpec(memory_space=pl.ANY),
        ],
        out_specs=pl.BlockSpec((BS, F), lambda p, i: (i, 0)),
        out_shape=jax.ShapeDtypeStruct((S, F), jnp.float32),
        scratch_shapes=[
            pltpu.VMEM((1, D), jnp.float32),
            pltpu.VMEM((D, F), jnp.float32),
            pltpu.VMEM((D, F), jnp.float32),
            pltpu.SMEM((TOPK,), jnp.float32),
            pltpu.SemaphoreType.DMA((2,)),
        ],
    )(x, W_gate_out, W_gate_in, W_gate_lin, W_experts)


# speedup vs baseline: 2.6493x; 2.3616x over previous
import jax
import jax.numpy as jnp
from jax.experimental import pallas as pl
from jax.experimental.pallas import tpu as pltpu

S, D, H, E, TOPK, F = 2048, 1024, 64, 16, 2, 64

BS = 512
NBLK = S // BS


def _moe_kernel(x_ref, wout_ref, wgin_t_ref, wglin_t_ref, wexp_t_hbm,
                out_ref, stash_ref, pad_ref, v_ref, w0t_ref, w1t_ref,
                gw_ref, sems):
    p = pl.program_id(0)
    i = pl.program_id(1)

    @pl.when(p == 0)
    def _gate_phase():
        @pl.when(i == 0)
        def _():
            v_ref[...] = jnp.zeros_like(v_ref)

        xb = x_ref[...]
        stash_ref[pl.ds(i * BS, BS), :] = xb
        v_ref[...] += jnp.dot(wout_ref[:, pl.ds(i * BS, BS)], xb,
                              preferred_element_type=jnp.float32)

        @pl.when(i == NBLK - 1)
        def _finish_gate():
            u = jax.lax.dot_general(
                v_ref[...], wgin_t_ref[...],
                dimension_numbers=(((1,), (1,)), ((), ())),
                preferred_element_type=jnp.float32)
            g = jax.lax.dot_general(
                u, wglin_t_ref[...],
                dimension_numbers=(((1,), (1,)), ((), ())),
                preferred_element_type=jnp.float32)
            iota = jax.lax.broadcasted_iota(jnp.int32, (1, E), 1)
            m0 = jnp.max(g)
            i0 = jnp.min(jnp.where(g == m0, iota, E))
            g2 = jnp.where(iota == i0, -jnp.inf, g)
            m1 = jnp.max(g2)
            i1 = jnp.min(jnp.where(g2 == m1, iota, E))
            e1 = jnp.exp(m1 - m0)
            denom = 1.0 + e1
            gw_ref[0] = 1.0 / denom
            gw_ref[1] = e1 / denom
            pltpu.make_async_copy(wexp_t_hbm.at[i0], w0t_ref,
                                  sems.at[0]).start()
            pltpu.make_async_copy(wexp_t_hbm.at[i1], w1t_ref,
                                  sems.at[1]).start()

    @pl.when(p == 1)
    def _expert_phase():
        @pl.when(i == 0)
        def _():
            pltpu.make_async_copy(wexp_t_hbm.at[0], w0t_ref, sems.at[0]).wait()
            pltpu.make_async_copy(wexp_t_hbm.at[0], w1t_ref, sems.at[1]).wait()

        xb = stash_ref[pl.ds(i * BS, BS), :]
        z0 = jax.lax.dot_general(
            xb, w0t_ref[...],
            dimension_numbers=(((1,), (1,)), ((), ())),
            preferred_element_type=jnp.float32)
        z1 = jax.lax.dot_general(
            xb, w1t_ref[...],
            dimension_numbers=(((1,), (1,)), ((), ())),
            preferred_element_type=jnp.float32)

        def norm_gelu(z):
            nrm = jnp.sqrt(jnp.sum(z * z, axis=-1, keepdims=True))
            zn = z / jnp.maximum(nrm, 1e-12)
            return 0.5 * zn * (1.0 + jax.lax.erf(zn * 0.7071067811865476))

        y = gw_ref[0] * norm_gelu(z0) + gw_ref[1] * norm_gelu(z1)
        out_ref[...] = y.T


@jax.jit
def kernel(x, W_gate_in, W_gate_lin, W_gate_out, W_experts):
    y_t = pl.pallas_call(
        _moe_kernel,
        grid=(2, NBLK),
        in_specs=[
            pl.BlockSpec((BS, D), lambda p, i: (i * (1 - p) + (NBLK - 1) * p, 0)),
            pl.BlockSpec((1, S), lambda p, i: (0, 0)),
            pl.BlockSpec((H, D), lambda p, i: (0, 0)),
            pl.BlockSpec((E, H), lambda p, i: (0, 0)),
            pl.BlockSpec(memory_space=pl.ANY),
        ],
        out_specs=pl.BlockSpec((F, BS), lambda p, i: (0, i)),
        out_shape=jax.ShapeDtypeStruct((F, S), jnp.float32),
        compiler_params=pltpu.CompilerParams(
            vmem_limit_bytes=100 * 1024 * 1024),
        scratch_shapes=[
            pltpu.VMEM((S, D), jnp.float32),
            pltpu.VMEM((11264, 1024), jnp.float32),
            pltpu.VMEM((1, D), jnp.float32),
            pltpu.VMEM((F, D), jnp.float32),
            pltpu.VMEM((F, D), jnp.float32),
            pltpu.SMEM((TOPK,), jnp.float32),
            pltpu.SemaphoreType.DMA((2,)),
        ],
    )(x, W_gate_out.reshape(1, S), W_gate_in.T, W_gate_lin.T,
      W_experts.transpose(0, 2, 1))
    return y_t.T


# speedup vs baseline: 3.0309x; 1.1441x over previous
import jax
import jax.numpy as jnp
from jax.experimental import pallas as pl
from jax.experimental.pallas import tpu as pltpu

S, D, H, E, TOPK, F = 2048, 1024, 64, 16, 2, 64

NC = 8
CS = S // NC


def _moe_kernel(x_hbm, wout_hbm, wgin_t_hbm, wglin_t_hbm, wexp_t_hbm,
                out_ref, stash_ref, pad_ref, wcat_ref, wout_ref, wgin_ref,
                wglin_ref, xsem, wsem, esem):
    for k in range(NC):
        pltpu.make_async_copy(
            x_hbm.at[pl.ds(k * CS, CS), :],
            stash_ref.at[pl.ds(k * CS, CS), :], xsem.at[k]).start()
    pltpu.make_async_copy(wout_hbm, wout_ref, wsem.at[0]).start()
    pltpu.make_async_copy(wgin_t_hbm, wgin_ref, wsem.at[1]).start()
    pltpu.make_async_copy(wglin_t_hbm, wglin_ref, wsem.at[2]).start()

    pltpu.make_async_copy(wout_hbm, wout_ref, wsem.at[0]).wait()
    v = jnp.zeros((1, D), dtype=jnp.float32)
    for k in range(NC):
        pltpu.make_async_copy(
            x_hbm.at[pl.ds(k * CS, CS), :],
            stash_ref.at[pl.ds(k * CS, CS), :], xsem.at[k]).wait()
        v = v + jnp.dot(wout_ref[:, k * CS:(k + 1) * CS],
                        stash_ref[k * CS:(k + 1) * CS, :],
                        preferred_element_type=jnp.float32)

    pltpu.make_async_copy(wgin_t_hbm, wgin_ref, wsem.at[1]).wait()
    pltpu.make_async_copy(wglin_t_hbm, wglin_ref, wsem.at[2]).wait()
    u = jax.lax.dot_general(v, wgin_ref[...],
                            dimension_numbers=(((1,), (1,)), ((), ())),
                            preferred_element_type=jnp.float32)
    g = jax.lax.dot_general(u, wglin_ref[...],
                            dimension_numbers=(((1,), (1,)), ((), ())),
                            preferred_element_type=jnp.float32)
    iota = jax.lax.broadcasted_iota(jnp.int32, (1, E), 1)
    m0 = jnp.max(g)
    i0 = jnp.min(jnp.where(g == m0, iota, E))
    g2 = jnp.where(iota == i0, -jnp.inf, g)
    m1 = jnp.max(g2)
    i1 = jnp.min(jnp.where(g2 == m1, iota, E))
    e1 = jnp.exp(m1 - m0)
    w0 = 1.0 / (1.0 + e1)
    w1 = e1 / (1.0 + e1)
    pltpu.make_async_copy(wexp_t_hbm.at[i0], wcat_ref.at[0:F, :],
                          esem.at[0]).start()
    pltpu.make_async_copy(wexp_t_hbm.at[i1], wcat_ref.at[F:2 * F, :],
                          esem.at[1]).start()
    pltpu.make_async_copy(wexp_t_hbm.at[0], wcat_ref.at[0:F, :],
                          esem.at[0]).wait()
    pltpu.make_async_copy(wexp_t_hbm.at[0], wcat_ref.at[F:2 * F, :],
                          esem.at[1]).wait()

    def norm_gelu(z):
        nrm = jnp.sqrt(jnp.sum(z * z, axis=-1, keepdims=True))
        zn = z / jnp.maximum(nrm, 1e-12)
        return 0.5 * zn * (1.0 + jax.lax.erf(zn * 0.7071067811865476))

    for k in range(NC):
        xb = stash_ref[k * CS:(k + 1) * CS, :]
        z = jax.lax.dot_general(xb, wcat_ref[...],
                                dimension_numbers=(((1,), (1,)), ((), ())),
                                preferred_element_type=jnp.float32)
        y = w0 * norm_gelu(z[:, 0:F]) + w1 * norm_gelu(z[:, F:2 * F])
        out_ref[:, k * CS:(k + 1) * CS] = y.T


@jax.jit
def kernel(x, W_gate_in, W_gate_lin, W_gate_out, W_experts):
    y_t = pl.pallas_call(
        _moe_kernel,
        in_specs=[
            pl.BlockSpec(memory_space=pl.ANY),
            pl.BlockSpec(memory_space=pl.ANY),
            pl.BlockSpec(memory_space=pl.ANY),
            pl.BlockSpec(memory_space=pl.ANY),
            pl.BlockSpec(memory_space=pl.ANY),
        ],
        out_specs=pl.BlockSpec((F, S), lambda: (0, 0)),
        out_shape=jax.ShapeDtypeStruct((F, S), jnp.float32),
        compiler_params=pltpu.CompilerParams(
            vmem_limit_bytes=100 * 1024 * 1024),
        scratch_shapes=[
            pltpu.VMEM((S, D), jnp.float32),
            pltpu.VMEM((11264, 1024), jnp.float32),
            pltpu.VMEM((2 * F, D), jnp.float32),
            pltpu.VMEM((1, S), jnp.float32),
            pltpu.VMEM((H, D), jnp.float32),
            pltpu.VMEM((E, H), jnp.float32),
            pltpu.SemaphoreType.DMA((NC,)),
            pltpu.SemaphoreType.DMA((3,)),
            pltpu.SemaphoreType.DMA((2,)),
        ],
    )(x, W_gate_out.reshape(1, S), W_gate_in.T, W_gate_lin.T,
      W_experts.transpose(0, 2, 1))
    return y_t.T
